# baseline (speedup 1.0000x reference)
import functools

import jax
import jax.numpy as jnp
from jax import lax
from jax.experimental import pallas as pl
from jax.experimental.pallas import tpu as pltpu
from jax.experimental.pallas import tpu_sc as plsc

NS = 8
LANES = 16


def _score_body(feat_ref, w_ref, out_ref):
    res = lax.dot_general(
        feat_ref[...], w_ref[...],
        dimension_numbers=(((1,), (1,)), ((), ())),
        precision=lax.Precision.HIGHEST,
        preferred_element_type=jnp.float32,
    )
    out_ref[...] = res[:, 0]


def _node_scores(features, w_pad, rows_per_block=1024):
    n = features.shape[0]
    dcols = w_pad.shape[1]
    grid = -(-n // rows_per_block)
    n_pad = grid * rows_per_block
    return pl.pallas_call(
        _score_body,
        grid=(grid,),
        in_specs=[
            pl.BlockSpec((rows_per_block, dcols), lambda i: (i, 0)),
            pl.BlockSpec((1, dcols), lambda i: (0, 0)),
        ],
        out_specs=pl.BlockSpec((rows_per_block,), lambda i: (i,)),
        out_shape=jax.ShapeDtypeStruct((n_pad,), jnp.float32),
    )(features, w_pad)


@functools.lru_cache(maxsize=None)
def _make_sampler(n_nodes, max_deg, n_ids, n_score):
    info = plsc.get_sparse_core_info()
    nc, nsub = info.num_cores, info.num_subcores
    nw = nc * nsub
    vpw = n_ids // nw
    ppw = vpw * NS
    gd = max_deg // NS
    assert gd == 8 and NS == 8 and n_ids % (nw * LANES) == 0
    idx_chunk = 128
    mesh = plsc.VectorSubcoreMesh(core_axis_name="c", subcore_axis_name="s")

    @functools.partial(
        pl.kernel,
        mesh=mesh,
        compiler_params=pltpu.CompilerParams(needs_layout_passes=False),
        out_type=[
            jax.ShapeDtypeStruct((n_ids * NS,), jnp.int32),
            jax.ShapeDtypeStruct((n_ids,), jnp.float32),
        ],
        scratch_types=[
            pltpu.VMEM((n_score,), jnp.float32),
            pltpu.VMEM((vpw,), jnp.int32),
            pltpu.VMEM((vpw,), jnp.int32),
            pltpu.VMEM((vpw, 2 * max_deg), jnp.int32),
            pltpu.VMEM((ppw,), jnp.int32),
            pltpu.VMEM((ppw,), jnp.float32),
            pltpu.VMEM((vpw,), jnp.float32),
            pltpu.SemaphoreType.DMA,
        ],
    )
    def sampler(score_hbm, ids_hbm, adj_hbm, sel_hbm, nnz_hbm,
                score_v, ids_v, ids2_v, adj_v, sel_v, nzi_v, nnz_v, sem):
        wid = lax.axis_index("s") * nc + lax.axis_index("c")
        vbase = wid * vpw
        pltpu.sync_copy(ids_hbm.at[pl.ds(vbase, vpw)], ids_v)
        for i in range(vpw // LANES):
            ids2_v[pl.ds(i * LANES, LANES)] = lax.shift_right_logical(
                ids_v[pl.ds(i * LANES, LANES)], 1)
        copies = [
            pltpu.async_copy(
                adj_hbm.at[ids2_v.at[pl.ds(c * idx_chunk, idx_chunk)]],
                adj_v.at[pl.ds(c * idx_chunk, idx_chunk)],
                sem,
            )
            for c in range(vpw // idx_chunk)
        ]
        pltpu.sync_copy(score_hbm, score_v)
        for cp in copies:
            cp.wait()

        sentinel = n_nodes - 1

        def pair_body(pb, carry):
            lanes = lax.iota(jnp.int32, LANES)
            p = pb * LANES + lanes
            row = lax.shift_right_logical(p, 3)
            idv = plsc.load_gather(ids_v, [row])
            colb = (lax.shift_left(idv & 1, 6)
                    + lax.shift_left(p & (NS - 1), 3))
            nbr = plsc.load_gather(adj_v, [row, colb])
            best_s = plsc.load_gather(score_v, [nbr])
            best_g = jnp.zeros((LANES,), jnp.int32)
            for g in range(1, gd):
                nbr = plsc.load_gather(adj_v, [row, colb + g])
                s = plsc.load_gather(score_v, [nbr])
                pred = s < best_s
                best_s = jnp.where(pred, s, best_s)
                best_g = jnp.where(pred, jnp.full((LANES,), g, jnp.int32), best_g)
            sel = plsc.load_gather(adj_v, [row, colb + best_g])
            sel_v[pl.ds(pb * LANES, LANES)] = sel
            nzi_v[pl.ds(pb * LANES, LANES)] = jnp.where(
                sel == sentinel,
                jnp.zeros((LANES,), jnp.float32),
                jnp.ones((LANES,), jnp.float32),
            )
            return carry

        lax.fori_loop(0, ppw // LANES, pair_body, None)

        def nnz_body(vb, carry):
            lanes = lax.iota(jnp.int32, LANES)
            v = vb * LANES + lanes
            vb8 = lax.shift_left(v, 3)
            acc = plsc.load_gather(nzi_v, [vb8])
            for g in range(1, NS):
                acc = acc + plsc.load_gather(nzi_v, [vb8 + g])
            nnz_v[pl.ds(vb * LANES, LANES)] = acc
            return carry

        lax.fori_loop(0, vpw // LANES, nnz_body, None)

        pltpu.sync_copy(sel_v, sel_hbm.at[pl.ds(wid * ppw, ppw)])
        pltpu.sync_copy(nnz_v, nnz_hbm.at[pl.ds(wid * vpw, vpw)])

    return sampler


def kernel(features, W, b, ids, num_samples, adj_info):
    n_nodes, _ = features.shape
    n_ids = ids.shape[0]
    max_deg = adj_info.shape[1]
    dcols = features.shape[1]

    w_pad = jnp.zeros((1, dcols), jnp.float32).at[0, :50].set(W[50:100, 0])
    scores = _node_scores(features, w_pad)

    adj_resh = adj_info.reshape(n_nodes // 2, 2 * max_deg)
    sel_flat, adj_numnz = _make_sampler(n_nodes, max_deg, n_ids, scores.shape[0])(
        scores, ids, adj_resh)
    adj_sel = sel_flat.reshape(n_ids, NS)
    att = jnp.ones((n_ids, NS), jnp.float32)
    return (adj_sel, att, adj_numnz, adj_numnz)


# speedup vs baseline: 1.0156x; 1.0156x over previous
import functools

import jax
import jax.numpy as jnp
from jax import lax
from jax.experimental import pallas as pl
from jax.experimental.pallas import tpu as pltpu
from jax.experimental.pallas import tpu_sc as plsc

NS = 8
LANES = 16


def _score_body(feat_ref, w_ref, out_ref):
    out_ref[...] = jnp.sum(feat_ref[...] * w_ref[...], axis=1)


def _node_scores(features, w_pad, rows_per_block=1024):
    n = features.shape[0]
    dcols = w_pad.shape[1]
    grid = -(-n // rows_per_block)
    n_pad = grid * rows_per_block
    return pl.pallas_call(
        _score_body,
        grid=(grid,),
        in_specs=[
            pl.BlockSpec((rows_per_block, dcols), lambda i: (i, 0)),
            pl.BlockSpec((1, dcols), lambda i: (0, 0)),
        ],
        out_specs=pl.BlockSpec((rows_per_block,), lambda i: (i,)),
        out_shape=jax.ShapeDtypeStruct((n_pad,), jnp.float32),
    )(features, w_pad)


@functools.lru_cache(maxsize=None)
def _make_sampler(n_nodes, max_deg, n_ids, n_score):
    info = plsc.get_sparse_core_info()
    nc, nsub = info.num_cores, info.num_subcores
    nw = nc * nsub
    vpw = n_ids // nw
    ppw = vpw * NS
    gd = max_deg // NS
    assert gd == 8 and NS == 8 and n_ids % (nw * LANES) == 0
    idx_chunk = 128
    mesh = plsc.VectorSubcoreMesh(core_axis_name="c", subcore_axis_name="s")

    @functools.partial(
        pl.kernel,
        mesh=mesh,
        compiler_params=pltpu.CompilerParams(
            needs_layout_passes=False, use_tc_tiling_on_sc=False),
        out_type=[
            jax.ShapeDtypeStruct((n_ids * NS,), jnp.int32),
            jax.ShapeDtypeStruct((n_ids,), jnp.float32),
        ],
        scratch_types=[
            pltpu.VMEM((n_score,), jnp.float32),
            pltpu.VMEM((vpw,), jnp.int32),
            pltpu.VMEM((vpw, max_deg), jnp.int32),
            pltpu.VMEM((ppw,), jnp.int32),
            pltpu.VMEM((ppw,), jnp.float32),
            pltpu.VMEM((vpw,), jnp.float32),
            pltpu.SemaphoreType.DMA,
        ],
    )
    def sampler(score_hbm, ids_hbm, adj_hbm, sel_hbm, nnz_hbm,
                score_v, ids_v, adj_v, sel_v, nzi_v, nnz_v, sem):
        wid = lax.axis_index("s") * nc + lax.axis_index("c")
        vbase = wid * vpw
        pltpu.sync_copy(ids_hbm.at[pl.ds(vbase, vpw)], ids_v)
        copies = [
            pltpu.async_copy(
                adj_hbm.at[ids_v.at[pl.ds(c * idx_chunk, idx_chunk)]],
                adj_v.at[pl.ds(c * idx_chunk, idx_chunk)],
                sem,
            )
            for c in range(vpw // idx_chunk)
        ]
        pltpu.sync_copy(score_hbm, score_v)
        for cp in copies:
            cp.wait()

        sentinel = n_nodes - 1

        def pair_body(pb, carry):
            lanes = lax.iota(jnp.int32, LANES)
            p = pb * LANES + lanes
            row = lax.shift_right_logical(p, 3)
            colb = lax.shift_left(p & (NS - 1), 3)
            nbr = plsc.load_gather(adj_v, [row, colb])
            best_s = plsc.load_gather(score_v, [nbr])
            best_g = jnp.zeros((LANES,), jnp.int32)
            for g in range(1, gd):
                nbr = plsc.load_gather(adj_v, [row, colb + g])
                s = plsc.load_gather(score_v, [nbr])
                pred = s < best_s
                best_s = jnp.where(pred, s, best_s)
                best_g = jnp.where(pred, jnp.full((LANES,), g, jnp.int32), best_g)
            sel = plsc.load_gather(adj_v, [row, colb + best_g])
            sel_v[pl.ds(pb * LANES, LANES)] = sel
            nzi_v[pl.ds(pb * LANES, LANES)] = jnp.where(
                sel == sentinel,
                jnp.zeros((LANES,), jnp.float32),
                jnp.ones((LANES,), jnp.float32),
            )
            return carry

        lax.fori_loop(0, ppw // LANES, pair_body, None)

        def nnz_body(vb, carry):
            lanes = lax.iota(jnp.int32, LANES)
            v = vb * LANES + lanes
            vb8 = lax.shift_left(v, 3)
            acc = plsc.load_gather(nzi_v, [vb8])
            for g in range(1, NS):
                acc = acc + plsc.load_gather(nzi_v, [vb8 + g])
            nnz_v[pl.ds(vb * LANES, LANES)] = acc
            return carry

        lax.fori_loop(0, vpw // LANES, nnz_body, None)

        pltpu.sync_copy(sel_v, sel_hbm.at[pl.ds(wid * ppw, ppw)])
        pltpu.sync_copy(nnz_v, nnz_hbm.at[pl.ds(wid * vpw, vpw)])

    return sampler


def kernel(features, W, b, ids, num_samples, adj_info):
    n_nodes, _ = features.shape
    n_ids = ids.shape[0]
    max_deg = adj_info.shape[1]
    dcols = features.shape[1]

    w_pad = jnp.zeros((1, dcols), jnp.float32).at[0, :50].set(W[50:100, 0])
    scores = _node_scores(features, w_pad)

    sel_flat, adj_numnz = _make_sampler(n_nodes, max_deg, n_ids, scores.shape[0])(
        scores, ids, adj_info)
    adj_sel = sel_flat.reshape(n_ids, NS)
    att = jnp.ones((n_ids, NS), jnp.float32)
    return (adj_sel, att, adj_numnz, adj_numnz)


# speedup vs baseline: 1.2003x; 1.1818x over previous
import functools

import jax
import jax.numpy as jnp
from jax import lax
from jax.experimental import pallas as pl
from jax.experimental.pallas import tpu as pltpu
from jax.experimental.pallas import tpu_sc as plsc

NS = 8
LANES = 16


def _score_body(feat_ref, w_ref, out_ref):
    out_ref[...] = jnp.sum(feat_ref[...] * w_ref[...], axis=1)


def _node_scores(features, w_pad, rows_per_block=8192):
    n = features.shape[0]
    dcols = w_pad.shape[1]
    grid = -(-n // rows_per_block)
    n_pad = grid * rows_per_block
    return pl.pallas_call(
        _score_body,
        grid=(grid,),
        in_specs=[
            pl.BlockSpec((rows_per_block, dcols), lambda i: (i, 0)),
            pl.BlockSpec((1, dcols), lambda i: (0, 0)),
        ],
        out_specs=pl.BlockSpec((rows_per_block,), lambda i: (i,)),
        out_shape=jax.ShapeDtypeStruct((n_pad,), jnp.float32),
    )(features, w_pad)


@functools.lru_cache(maxsize=None)
def _make_sampler(n_nodes, max_deg, n_ids, n_score):
    info = plsc.get_sparse_core_info()
    nc, nsub = info.num_cores, info.num_subcores
    nw = nc * nsub
    vpw = n_ids // nw
    ppw = vpw * NS
    gd = max_deg // NS
    assert gd == 8 and NS == 8 and n_ids % (nw * LANES) == 0
    idx_chunk = 128
    mesh = plsc.VectorSubcoreMesh(core_axis_name="c", subcore_axis_name="s")

    @functools.partial(
        pl.kernel,
        mesh=mesh,
        compiler_params=pltpu.CompilerParams(
            needs_layout_passes=False, use_tc_tiling_on_sc=False),
        out_type=[
            jax.ShapeDtypeStruct((n_ids * NS,), jnp.int32),
            jax.ShapeDtypeStruct((n_ids,), jnp.float32),
        ],
        scratch_types=[
            pltpu.VMEM((n_score,), jnp.float32),
            pltpu.VMEM((vpw,), jnp.int32),
            pltpu.VMEM((vpw, max_deg), jnp.int32),
            pltpu.VMEM((ppw,), jnp.int32),
            pltpu.VMEM((ppw,), jnp.float32),
            pltpu.VMEM((vpw,), jnp.float32),
            pltpu.SemaphoreType.DMA,
        ],
    )
    def sampler(score_hbm, ids_hbm, adj_hbm, sel_hbm, nnz_hbm,
                score_v, ids_v, adj_v, sel_v, nzi_v, nnz_v, sem):
        wid = lax.axis_index("s") * nc + lax.axis_index("c")
        vbase = wid * vpw
        pltpu.sync_copy(ids_hbm.at[pl.ds(vbase, vpw)], ids_v)
        copies = [
            pltpu.async_copy(
                adj_hbm.at[ids_v.at[pl.ds(c * idx_chunk, idx_chunk)]],
                adj_v.at[pl.ds(c * idx_chunk, idx_chunk)],
                sem,
            )
            for c in range(vpw // idx_chunk)
        ]
        pltpu.sync_copy(score_hbm, score_v)
        for cp in copies:
            cp.wait()

        sentinel = n_nodes - 1

        def pair_body(pb, carry):
            lanes = lax.iota(jnp.int32, LANES)
            p = pb * LANES + lanes
            row = lax.shift_right_logical(p, 3)
            colb = lax.shift_left(p & (NS - 1), 3)
            nbr = plsc.load_gather(adj_v, [row, colb])
            best_s = plsc.load_gather(score_v, [nbr])
            best_g = jnp.zeros((LANES,), jnp.int32)
            for g in range(1, gd):
                nbr = plsc.load_gather(adj_v, [row, colb + g])
                s = plsc.load_gather(score_v, [nbr])
                pred = s < best_s
                best_s = jnp.where(pred, s, best_s)
                best_g = jnp.where(pred, jnp.full((LANES,), g, jnp.int32), best_g)
            sel = plsc.load_gather(adj_v, [row, colb + best_g])
            sel_v[pl.ds(pb * LANES, LANES)] = sel
            nzi_v[pl.ds(pb * LANES, LANES)] = jnp.where(
                sel == sentinel,
                jnp.zeros((LANES,), jnp.float32),
                jnp.ones((LANES,), jnp.float32),
            )
            return carry

        lax.fori_loop(0, ppw // LANES, pair_body, None)

        def nnz_body(vb, carry):
            lanes = lax.iota(jnp.int32, LANES)
            v = vb * LANES + lanes
            vb8 = lax.shift_left(v, 3)
            acc = plsc.load_gather(nzi_v, [vb8])
            for g in range(1, NS):
                acc = acc + plsc.load_gather(nzi_v, [vb8 + g])
            nnz_v[pl.ds(vb * LANES, LANES)] = acc
            return carry

        lax.fori_loop(0, vpw // LANES, nnz_body, None)

        pltpu.sync_copy(sel_v, sel_hbm.at[pl.ds(wid * ppw, ppw)])
        pltpu.sync_copy(nnz_v, nnz_hbm.at[pl.ds(wid * vpw, vpw)])

    return sampler


def kernel(features, W, b, ids, num_samples, adj_info):
    n_nodes, _ = features.shape
    n_ids = ids.shape[0]
    max_deg = adj_info.shape[1]
    dcols = features.shape[1]

    w_pad = jnp.zeros((1, dcols), jnp.float32).at[0, :50].set(W[50:100, 0])
    scores = _node_scores(features, w_pad)

    sel_flat, adj_numnz = _make_sampler(n_nodes, max_deg, n_ids, scores.shape[0])(
        scores, ids, adj_info)
    adj_sel = sel_flat.reshape(n_ids, NS)
    att = jnp.ones((n_ids, NS), jnp.float32)
    return (adj_sel, att, adj_numnz, adj_numnz)
